# baseline (speedup 1.0000x reference)
import functools

import jax
import jax.numpy as jnp
from jax import lax
from jax.experimental import pallas as pl
from jax.experimental.pallas import tpu as pltpu
from jax.experimental.pallas import tpu_sc as plsc

_NC = 2
_NS = 16
_NW = _NC * _NS
_L = 16

_EPS = 1e-12


def _rsqrt_scalar(v):
    i = lax.bitcast_convert_type(v, jnp.int32)
    i = jnp.int32(0x5F3759DF) - (i >> 1)
    y = lax.bitcast_convert_type(i, jnp.float32)
    for _ in range(2):
        y = y * (1.5 - 0.5 * v * y * y)
    return y


def _build_sc_call(B, S, H, VOCAB, NBINS, S_PAD):
    rows_per_w = B // _NW
    n_chunks = 2
    chunk = S // n_chunks
    nj = H // _L
    mesh = plsc.VectorSubcoreMesh(
        core_axis_name="c", subcore_axis_name="s",
        num_cores=_NC, num_subcores=_NS)

    @functools.partial(
        pl.kernel,
        out_type=jax.ShapeDtypeStruct((B, S, H), jnp.float32),
        mesh=mesh,
        compiler_params=pltpu.CompilerParams(needs_layout_passes=False),
        scratch_types=[
            pltpu.VMEM((rows_per_w, n_chunks, chunk), jnp.int32),
            pltpu.VMEM((rows_per_w, S_PAD), jnp.int32),
            pltpu.VMEM((S, H), jnp.float32),
            pltpu.VMEM((S, H), jnp.float32),
            pltpu.VMEM((S, H), jnp.float32),
            pltpu.VMEM((NBINS, H), jnp.float32),
            pltpu.VMEM((S, H), jnp.float32),
            pltpu.VMEM((2, H), jnp.float32),
            pltpu.SemaphoreType.DMA,
            pltpu.SemaphoreType.DMA,
            pltpu.SemaphoreType.DMA,
            pltpu.SemaphoreType.DMA,
            pltpu.SemaphoreType.DMA,
            pltpu.SemaphoreType.DMA,
        ],
    )
    def sc_kernel(ids_hbm, bins_hbm, gene_hbm, expr_hbm, pos_hbm, gam_hbm,
                  bet_hbm, out_hbm, ids_v, bins_v, buf0, buf1, buf2,
                  expr_v, pos_v, gb_v, g0, g1, g2, o0, o1, o2):
        wid = lax.axis_index("s") * _NC + lax.axis_index("c")
        base = wid * rows_per_w
        bufs = [buf0, buf1, buf2]
        gsems = [g0, g1, g2]
        osems = [o0, o1, o2]

        pltpu.sync_copy(ids_hbm.at[pl.ds(base, rows_per_w)], ids_v)
        pltpu.sync_copy(bins_hbm.at[pl.ds(base, rows_per_w)], bins_v)
        pltpu.sync_copy(expr_hbm, expr_v)
        pltpu.sync_copy(pos_hbm.at[pl.ds(0, S)], pos_v)
        pltpu.sync_copy(gam_hbm, gb_v.at[0])
        pltpu.sync_copy(bet_hbm, gb_v.at[1])

        gams = [gb_v[0, pl.ds(16 * j, 16)] for j in range(nj)]
        bets = [gb_v[1, pl.ds(16 * j, 16)] for j in range(nj)]
        invh = jnp.float32(1.0 / H)

        def start_gather(b, rloc):
            for k in range(n_chunks):
                pltpu.async_copy(
                    gene_hbm.at[ids_v.at[rloc, k]],
                    bufs[b].at[pl.ds(k * chunk, chunk)], gsems[b])

        def wait_gather(b, rloc):
            for k in range(n_chunks):
                pltpu.make_async_copy(
                    gene_hbm.at[ids_v.at[rloc, k]],
                    bufs[b].at[pl.ds(k * chunk, chunk)], gsems[b]).wait()

        def start_out(b, rloc):
            pltpu.async_copy(bufs[b], out_hbm.at[base + rloc], osems[b])

        def wait_out(b, rloc):
            pltpu.make_async_copy(
                bufs[b], out_hbm.at[base + rloc], osems[b]).wait()

        off16 = lax.iota(jnp.int32, _L)
        offs = [off16 + jnp.int32(16 * j) for j in range(nj)]

        def token(buf, s, bin_s):
            xs = []
            s1 = None
            s2 = None
            for j in range(nj):
                ev = plsc.load_gather(expr_v, [bin_s, offs[j]])
                gv = buf[s, pl.ds(16 * j, 16)]
                pv = pos_v[s, pl.ds(16 * j, 16)]
                x = gv + ev + pv
                xs.append(x)
                s1 = x if s1 is None else s1 + x
                s2 = x * x if s2 is None else s2 + x * x
            mean_s = jnp.sum(s1) * invh
            var_s = jnp.sum(s2) * invh - mean_s * mean_s + jnp.float32(_EPS)
            iv_s = _rsqrt_scalar(var_s)
            inv = jnp.full((_L,), iv_s, dtype=jnp.float32)
            mean = jnp.full((_L,), mean_s, dtype=jnp.float32)
            for j in range(nj):
                y = (xs[j] - mean) * inv
                buf[s, pl.ds(16 * j, 16)] = y * gams[j] + bets[j]

        def compute(b, rloc):
            buf = bufs[b]
            usplat = jnp.full((_L,), rloc, dtype=jnp.int32)

            @plsc.parallel_loop(0, S, step=1, unroll=1)
            def _body(s):
                ssplat = jnp.full((_L,), s, dtype=jnp.int32)
                binv = plsc.load_gather(bins_v, [usplat, ssplat])
                token(buf, s, binv)

        start_gather(0, 0)

        def pipe3(k, c):
            for jph in range(3):
                r = 3 * k + jph

                @pl.when(r < rows_per_w)
                def _():
                    wait_gather(jph, r)

                    @pl.when(jnp.logical_and(r >= 2, r + 1 < rows_per_w))
                    def _():
                        wait_out((jph + 1) % 3, r - 2)

                    @pl.when(r + 1 < rows_per_w)
                    def _():
                        start_gather((jph + 1) % 3, r + 1)

                    compute(jph, r)
                    start_out(jph, r)
            return c

        lax.fori_loop(0, (rows_per_w + 2) // 3, pipe3, 0, unroll=False)
        wait_out((rows_per_w - 3) % 3, rows_per_w - 3)
        wait_out((rows_per_w - 2) % 3, rows_per_w - 2)
        wait_out((rows_per_w - 1) % 3, rows_per_w - 1)

    return sc_kernel


def kernel(gene_ids, expression_bins, gene_table, expr_table, pos_table,
           ln_gamma, ln_beta):
    B, S = gene_ids.shape
    VOCAB, H = gene_table.shape
    NBINS = expr_table.shape[0]
    ids2 = gene_ids.reshape(B, 2, S // 2)
    s_pad = ((S + _L - 1) // _L) * _L
    bins_p = jnp.pad(expression_bins, ((0, 0), (0, s_pad - S)))
    fn = _build_sc_call(B, S, H, VOCAB, NBINS, s_pad)
    return fn(ids2, bins_p, gene_table, expr_table, pos_table,
              ln_gamma, ln_beta)


# speedup vs baseline: 1.9804x; 1.9804x over previous
import functools

import jax
import jax.numpy as jnp
from jax import lax
from jax.experimental import pallas as pl
from jax.experimental.pallas import tpu as pltpu
from jax.experimental.pallas import tpu_sc as plsc

_NC = 2
_NS = 16
_NW = _NC * _NS
_L = 16

_EPS = 1e-12


def _rsqrt_scalar(v):
    i = lax.bitcast_convert_type(v, jnp.int32)
    i = jnp.int32(0x5F3759DF) - (i >> 1)
    y = lax.bitcast_convert_type(i, jnp.float32)
    for _ in range(2):
        y = y * (1.5 - 0.5 * v * y * y)
    return y


def _build_sc_call(B, S, H, VOCAB, NBINS, S_PAD):
    rows_per_w = B // _NW
    n_chunks = 2
    chunk = S // n_chunks
    nj = H // _L
    mesh = plsc.VectorSubcoreMesh(
        core_axis_name="c", subcore_axis_name="s",
        num_cores=_NC, num_subcores=_NS)

    @functools.partial(
        pl.kernel,
        out_type=jax.ShapeDtypeStruct((B, S, H), jnp.float32),
        mesh=mesh,
        compiler_params=pltpu.CompilerParams(needs_layout_passes=False),
        scratch_types=[
            pltpu.VMEM((rows_per_w, n_chunks, chunk), jnp.int32),
            pltpu.VMEM((rows_per_w, S_PAD), jnp.int32),
            pltpu.VMEM((S, H), jnp.float32),
            pltpu.VMEM((S, H), jnp.float32),
            pltpu.VMEM((S, H), jnp.float32),
            pltpu.VMEM((NBINS, H), jnp.float32),
            pltpu.VMEM((S, H), jnp.float32),
            pltpu.VMEM((2, H), jnp.float32),
            pltpu.SemaphoreType.DMA,
            pltpu.SemaphoreType.DMA,
            pltpu.SemaphoreType.DMA,
            pltpu.SemaphoreType.DMA,
            pltpu.SemaphoreType.DMA,
            pltpu.SemaphoreType.DMA,
        ],
    )
    def sc_kernel(ids_hbm, bins_hbm, gene_hbm, expr_hbm, pos_hbm, gam_hbm,
                  bet_hbm, out_hbm, ids_v, bins_v, buf0, buf1, buf2,
                  expr_v, pos_v, gb_v, g0, g1, g2, o0, o1, o2):
        wid = lax.axis_index("s") * _NC + lax.axis_index("c")
        base = wid * rows_per_w
        bufs = [buf0, buf1, buf2]
        gsems = [g0, g1, g2]
        osems = [o0, o1, o2]

        pltpu.sync_copy(ids_hbm.at[pl.ds(base, rows_per_w)], ids_v)
        pltpu.sync_copy(bins_hbm.at[pl.ds(base, rows_per_w)], bins_v)
        pltpu.sync_copy(expr_hbm, expr_v)
        pltpu.sync_copy(pos_hbm.at[pl.ds(0, S)], pos_v)
        pltpu.sync_copy(gam_hbm, gb_v.at[0])
        pltpu.sync_copy(bet_hbm, gb_v.at[1])

        gams = [gb_v[0, pl.ds(16 * j, 16)] for j in range(nj)]
        bets = [gb_v[1, pl.ds(16 * j, 16)] for j in range(nj)]
        invh = jnp.float32(1.0 / H)

        def start_gather(b, rloc):
            for k in range(n_chunks):
                pltpu.async_copy(
                    gene_hbm.at[ids_v.at[rloc, k]],
                    bufs[b].at[pl.ds(k * chunk, chunk)], gsems[b])

        def wait_gather(b, rloc):
            for k in range(n_chunks):
                pltpu.make_async_copy(
                    gene_hbm.at[ids_v.at[rloc, k]],
                    bufs[b].at[pl.ds(k * chunk, chunk)], gsems[b]).wait()

        def start_out(b, rloc):
            pltpu.async_copy(bufs[b], out_hbm.at[base + rloc], osems[b])

        def wait_out(b, rloc):
            pltpu.make_async_copy(
                bufs[b], out_hbm.at[base + rloc], osems[b]).wait()

        def token(buf, s, bin_s):
            xs = []
            s1 = None
            s2 = None
            for j in range(nj):
                ev = expr_v[bin_s, pl.ds(16 * j, 16)]
                gv = buf[s, pl.ds(16 * j, 16)]
                pv = pos_v[s, pl.ds(16 * j, 16)]
                x = gv + ev + pv
                xs.append(x)
                s1 = x if s1 is None else s1 + x
                s2 = x * x if s2 is None else s2 + x * x
            mean_s = jnp.sum(s1) * invh
            var_s = jnp.sum(s2) * invh - mean_s * mean_s + jnp.float32(_EPS)
            iv_s = _rsqrt_scalar(var_s)
            inv = jnp.full((_L,), iv_s, dtype=jnp.float32)
            mean = jnp.full((_L,), mean_s, dtype=jnp.float32)
            for j in range(nj):
                y = (xs[j] - mean) * inv
                buf[s, pl.ds(16 * j, 16)] = y * gams[j] + bets[j]

        def compute(b, rloc):
            buf = bufs[b]
            usplat = jnp.full((_L,), rloc, dtype=jnp.int32)

            @plsc.parallel_loop(0, S, step=1, unroll=1)
            def _body(s):
                ssplat = jnp.full((_L,), s, dtype=jnp.int32)
                binv = plsc.load_gather(bins_v, [usplat, ssplat])
                token(buf, s, binv[0])

        start_gather(0, 0)

        def pipe3(k, c):
            for jph in range(3):
                r = 3 * k + jph

                @pl.when(r < rows_per_w)
                def _():
                    wait_gather(jph, r)

                    @pl.when(jnp.logical_and(r >= 2, r + 1 < rows_per_w))
                    def _():
                        wait_out((jph + 1) % 3, r - 2)

                    @pl.when(r + 1 < rows_per_w)
                    def _():
                        start_gather((jph + 1) % 3, r + 1)

                    compute(jph, r)
                    start_out(jph, r)
            return c

        lax.fori_loop(0, (rows_per_w + 2) // 3, pipe3, 0, unroll=False)
        wait_out((rows_per_w - 3) % 3, rows_per_w - 3)
        wait_out((rows_per_w - 2) % 3, rows_per_w - 2)
        wait_out((rows_per_w - 1) % 3, rows_per_w - 1)

    return sc_kernel


def kernel(gene_ids, expression_bins, gene_table, expr_table, pos_table,
           ln_gamma, ln_beta):
    B, S = gene_ids.shape
    VOCAB, H = gene_table.shape
    NBINS = expr_table.shape[0]
    ids2 = gene_ids.reshape(B, 2, S // 2)
    s_pad = ((S + _L - 1) // _L) * _L
    bins_p = jnp.pad(expression_bins, ((0, 0), (0, s_pad - S)))
    fn = _build_sc_call(B, S, H, VOCAB, NBINS, s_pad)
    return fn(ids2, bins_p, gene_table, expr_table, pos_table,
              ln_gamma, ln_beta)
